# Initial kernel scaffold; baseline (speedup 1.0000x reference)
import jax
import jax.numpy as jnp
from jax.experimental import pallas as pl


def kernel(x, row, col, edge_weight, W, b):
    raise NotImplementedError("write your pallas kernel here")


# speedup vs baseline: 3.7090x; 3.7090x over previous
import functools

import jax
import jax.numpy as jnp
from jax import lax
from jax.experimental import pallas as pl
from jax.experimental.pallas import tpu as pltpu
from jax.experimental.pallas import tpu_sc as plsc

N = 10000
E = 320000
D = 128
NC = 2
NS = 16
NW = NC * NS
B = 128
NBLK = -(-E // (NW * B))
EPAD = NW * NBLK * B
ROWS_PER_SUB = (N // NS) // 8 * 8
ROWS_REM = N - ROWS_PER_SUB * NS


def _matmul_body(x_ref, w_ref, o_ref):
    o_ref[...] = jnp.dot(x_ref[...], w_ref[...],
                         preferred_element_type=jnp.float32)


def _matmul(x, W):
    BM = 400
    return pl.pallas_call(
        _matmul_body,
        grid=(N // BM,),
        in_specs=[pl.BlockSpec((BM, D), lambda i: (i, 0)),
                  pl.BlockSpec((D, D), lambda i: (0, 0))],
        out_specs=pl.BlockSpec((BM, D), lambda i: (i, 0)),
        out_shape=jax.ShapeDtypeStruct((N, D), jnp.float32),
    )(x, W)


@functools.partial(
    pl.kernel,
    mesh=plsc.VectorSubcoreMesh(core_axis_name="c", subcore_axis_name="s"),
    out_type=jax.ShapeDtypeStruct((NC, N, D), jnp.float32),
    scratch_types=[
        pltpu.VMEM((B,), jnp.int32),
        pltpu.VMEM((B,), jnp.int32),
        pltpu.VMEM((B,), jnp.float32),
        pltpu.VMEM((B, D), jnp.float32),
        pltpu.VMEM_SHARED((N, D), jnp.float32),
        pltpu.SemaphoreType.DMA,
    ],
)
def _sc_agg(support, col2d, row2d, w2d, zeros, out,
            colv, rowv, wv, rowsbuf, acc, sem):
    core = lax.axis_index("c")
    sub = lax.axis_index("s")
    wid = core * NS + sub

    pltpu.sync_copy(zeros.at[pl.ds(sub * ROWS_PER_SUB, ROWS_PER_SUB)],
                    acc.at[pl.ds(sub * ROWS_PER_SUB, ROWS_PER_SUB)])

    @pl.when(sub == 0)
    def _zero_rem():
        pltpu.sync_copy(zeros.at[pl.ds(NS * ROWS_PER_SUB, ROWS_REM)],
                        acc.at[pl.ds(NS * ROWS_PER_SUB, ROWS_REM)])

    plsc.subcore_barrier()

    base_blk = wid * NBLK

    def blk_body(blk, carry):
        i = base_blk + blk
        pltpu.sync_copy(col2d.at[i], colv)
        pltpu.sync_copy(row2d.at[i], rowv)
        pltpu.sync_copy(w2d.at[i], wv)
        pltpu.async_copy(support.at[colv], rowsbuf, sem).wait()

        def g_body(g, c):
            w16 = wv[pl.ds(g * 16, 16)]
            for l in range(16):
                w = w16[l]
                e = g * 16 + l
                for j in range(D // 16):
                    sl = pl.ds(j * 16, 16)
                    rowsbuf[e, sl] = rowsbuf[e, sl] * w
            return c
        lax.fori_loop(0, B // 16, g_body, 0)

        pltpu.sync_copy(rowsbuf, acc.at[rowv], add=True)
        return carry

    lax.fori_loop(0, NBLK, blk_body, 0)
    plsc.subcore_barrier()

    pltpu.sync_copy(acc.at[pl.ds(sub * ROWS_PER_SUB, ROWS_PER_SUB)],
                    out.at[core, pl.ds(sub * ROWS_PER_SUB, ROWS_PER_SUB)])

    @pl.when(sub == 0)
    def _out_rem():
        pltpu.sync_copy(acc.at[pl.ds(NS * ROWS_PER_SUB, ROWS_REM)],
                        out.at[core, pl.ds(NS * ROWS_PER_SUB, ROWS_REM)])


def _combine_body(p_ref, b_ref, o_ref):
    o_ref[...] = p_ref[0] + p_ref[1] + b_ref[...]


def _combine(partials, b2d):
    BM = 400
    return pl.pallas_call(
        _combine_body,
        grid=(N // BM,),
        in_specs=[pl.BlockSpec((NC, BM, D), lambda i: (0, i, 0)),
                  pl.BlockSpec((1, D), lambda i: (0, 0))],
        out_specs=pl.BlockSpec((BM, D), lambda i: (i, 0)),
        out_shape=jax.ShapeDtypeStruct((N, D), jnp.float32),
    )(partials, b2d)


def kernel(x, row, col, edge_weight, W, b):
    support = _matmul(x, W)
    pad = EPAD - E
    col_p = jnp.pad(col.astype(jnp.int32), (0, pad)).reshape(NW * NBLK, B)
    row_p = jnp.pad(row.astype(jnp.int32), (0, pad)).reshape(NW * NBLK, B)
    w_p = jnp.pad(edge_weight.astype(jnp.float32), (0, pad)).reshape(
        NW * NBLK, B)
    zeros = jnp.zeros((N, D), jnp.float32)
    partials = _sc_agg(support, col_p, row_p, w_p, zeros)
    return _combine(partials, b.reshape(1, D))
